# Initial kernel scaffold; baseline (speedup 1.0000x reference)
import jax
import jax.numpy as jnp
from jax.experimental import pallas as pl


def kernel(x, table):
    raise NotImplementedError("write your pallas kernel here")


# speedup vs baseline: 4.9725x; 4.9725x over previous
import functools

import jax
import jax.numpy as jnp
from jax import lax
from jax.experimental import pallas as pl
from jax.experimental.pallas import tpu as pltpu
from jax.experimental.pallas import tpu_sc as plsc

_B, _S = 4096, 200
_D = 64
_N = _B * _S
_IW = 128
_IROWS = _N // _IW

_NC, _NS = 2, 16
_NW = _NC * _NS
_ROWS_PER_W = _IROWS // _NW
_CHUNK_IR = 4
_CHUNK = _CHUNK_IR * _IW
_NCHUNK = _ROWS_PER_W // _CHUNK_IR
_NBUF = 2


def _gather_body(table_hbm, idx_hbm, out_hbm, idx_v, buf0, buf1, g0, g1, w0, w1):
    wid = lax.axis_index("s") * _NC + lax.axis_index("c")
    row0 = wid * _ROWS_PER_W
    out0 = row0 * _IW

    bufs = (buf0, buf1)
    gsems = (g0, g1)
    wsems = (w0, w1)

    pltpu.sync_copy(idx_hbm.at[pl.ds(row0, _ROWS_PER_W)], idx_v)

    def fire_gathers(c, b):
        for j in range(_CHUNK_IR):
            pltpu.async_copy(
                table_hbm.at[idx_v.at[c * _CHUNK_IR + j]],
                bufs[b].at[pl.ds(j * _IW, _IW)],
                gsems[b],
            )

    def wait_gathers(c, b):
        for j in range(_CHUNK_IR):
            pltpu.make_async_copy(
                table_hbm.at[idx_v.at[c * _CHUNK_IR + j]],
                bufs[b].at[pl.ds(j * _IW, _IW)],
                gsems[b],
            ).wait()

    def fire_write(c, b):
        pltpu.async_copy(
            bufs[b], out_hbm.at[pl.ds(out0 + c * _CHUNK, _CHUNK)], wsems[b]
        )

    def wait_write(c, b):
        pltpu.make_async_copy(
            bufs[b], out_hbm.at[pl.ds(out0 + c * _CHUNK, _CHUNK)], wsems[b]
        ).wait()

    for b in range(_NBUF):
        fire_gathers(b, b)

    @pl.loop(0, _NCHUNK - _NBUF, step=_NBUF)
    def _steady(c0):
        for b in range(_NBUF):
            c = c0 + b
            wait_gathers(c, b)
            fire_write(c, b)
            wait_write(c, b)
            fire_gathers(c + _NBUF, b)

    for b in range(_NBUF):
        c = _NCHUNK - _NBUF + b
        wait_gathers(c, b)
        fire_write(c, b)
        wait_write(c, b)


@jax.jit
def _sc_gather(table, idx2d):
    mesh = plsc.VectorSubcoreMesh(core_axis_name="c", subcore_axis_name="s")
    run = pl.kernel(
        _gather_body,
        out_type=jax.ShapeDtypeStruct((_N, _D), jnp.float32),
        mesh=mesh,
        scratch_types=[
            pltpu.VMEM((_ROWS_PER_W, _IW), jnp.int32),
            pltpu.VMEM((_CHUNK, _D), jnp.float32),
            pltpu.VMEM((_CHUNK, _D), jnp.float32),
            pltpu.SemaphoreType.DMA,
            pltpu.SemaphoreType.DMA,
            pltpu.SemaphoreType.DMA,
            pltpu.SemaphoreType.DMA,
        ],
        compiler_params=pltpu.CompilerParams(use_tc_tiling_on_sc=False),
    )
    return run(table, idx2d)


def kernel(x, table):
    idx2d = x.reshape(_IROWS, _IW)
    out = _sc_gather(table, idx2d)
    return out.reshape(_B, _S, _D)
